# baseline (speedup 1.0000x reference)
import functools

import jax
import jax.numpy as jnp
from jax.experimental import pallas as pl
from jax.experimental.pallas import tpu as pltpu

EPS = 1e-5
V = 25
K = 3
NTAP = 9
PAD = 4
NB = 2


def _finalize(st_ref, g_ref, be_ref, cnt):
    tot = jnp.sum(st_ref[...], axis=0)
    mean = tot[:, 0:1] / cnt
    var = tot[:, 1:2] / cnt - mean * mean
    scale = g_ref[...] * jax.lax.rsqrt(var + EPS)
    shift = be_ref[...] - mean * scale
    return scale, shift


def _row_stats(val, gm_ref, sto_ref):
    s1 = jnp.sum(val, axis=1, keepdims=True)
    s2 = jnp.sum(val * val, axis=1, keepdims=True)
    g = gm_ref[...]
    sto_ref[0, :, 0:1] = jnp.dot(g, s1, preferred_element_type=jnp.float32)
    sto_ref[0, :, 1:2] = jnp.dot(g, s2, preferred_element_type=jnp.float32)


def _stats_in_body(x_ref, g_ref, o_ref):
    x = x_ref[...]
    s1 = jnp.sum(x, axis=1, keepdims=True)
    s2 = jnp.sum(x * x, axis=1, keepdims=True)
    g = g_ref[...]
    o_ref[:, 0:1] = jnp.dot(g, s1, preferred_element_type=jnp.float32)
    o_ref[:, 1:2] = jnp.dot(g, s2, preferred_element_type=jnp.float32)


def _sgc0_body(x_ref, st_ref, g0_ref, b0_ref, m_ref, b_ref, gm_ref,
               u_ref, sto_ref, *, nbatch):
    t = x_ref.shape[2]
    cnt = jnp.float32(nbatch * t * V)
    tot = st_ref[...]
    mean = tot[:, 0:1] / cnt
    var = tot[:, 1:2] / cnt - mean * mean
    scale = g0_ref[...] * jax.lax.rsqrt(var + EPS)
    shift = b0_ref[...] - mean * scale
    s_rows = jnp.concatenate([scale] * V, axis=0)
    t_rows = jnp.concatenate([shift] * V, axis=0)
    xn = jnp.concatenate([x_ref[j] * s_rows + t_rows for j in range(NB)],
                         axis=1)
    u = jnp.dot(m_ref[...], xn, preferred_element_type=jnp.float32)
    u = u + b_ref[...]
    for j in range(NB):
        u_ref[j] = u[:, j * t:(j + 1) * t]
    _row_stats(u, gm_ref, sto_ref)


def _tconv_core(u_ref, scale, shift, w_ref, bt_ref, gap_ref, hp_ref):
    c = w_ref.shape[0]
    t = u_ref.shape[2]
    tp = t + 2 * PAD
    vtp = V * tp
    w_all = NB * vtp
    hp_ref[...] = jnp.zeros((c, w_all + 2 * PAD), jnp.float32)
    for j in range(NB):
        for vv_ in range(V):
            blk = u_ref[j, vv_ * c:(vv_ + 1) * c, :]
            h = jnp.maximum(blk * scale + shift, 0.0)
            base = 2 * PAD + j * vtp + vv_ * tp
            hp_ref[:, base:base + t] = h
    acc = jnp.zeros((c, w_all), jnp.float32)
    for dt in range(NTAP):
        acc = acc + jnp.dot(w_ref[:, dt * c:(dt + 1) * c],
                            hp_ref[:, dt:dt + w_all],
                            preferred_element_type=jnp.float32)
    out = acc + bt_ref[...]
    return out, out * gap_ref[...]


def _tconv_body(u_ref, st_ref, g_ref, be_ref, w_ref, bt_ref, gap_ref,
                v_ref, sto_ref, hp_ref):
    t = u_ref.shape[2]
    vtp = V * (t + 2 * PAD)
    cnt = jnp.float32(st_ref.shape[0] * NB * t * V)
    scale, shift = _finalize(st_ref, g_ref, be_ref, cnt)
    out, outm = _tconv_core(u_ref, scale, shift, w_ref, bt_ref, gap_ref, hp_ref)
    for j in range(NB):
        v_ref[j] = out[:, j * vtp:(j + 1) * vtp]
    sto_ref[0, :, 0:1] = jnp.sum(outm, axis=1, keepdims=True)
    sto_ref[0, :, 1:2] = jnp.sum(outm * outm, axis=1, keepdims=True)


def _tconv_proj_body(u_ref, st_ref, g_ref, be_ref, w_ref, bt_ref, gap_ref,
                     x_ref, wr_ref, br_ref,
                     v_ref, sto_ref, r_ref, str_ref, hp_ref):
    t = u_ref.shape[2]
    vtp = V * (t + 2 * PAD)
    cnt = jnp.float32(st_ref.shape[0] * NB * t * V)
    scale, shift = _finalize(st_ref, g_ref, be_ref, cnt)
    out, outm = _tconv_core(u_ref, scale, shift, w_ref, bt_ref, gap_ref, hp_ref)
    for j in range(NB):
        v_ref[j] = out[:, j * vtp:(j + 1) * vtp]
    sto_ref[0, :, 0:1] = jnp.sum(outm, axis=1, keepdims=True)
    sto_ref[0, :, 1:2] = jnp.sum(outm * outm, axis=1, keepdims=True)
    xcat = jnp.concatenate([x_ref[j] for j in range(NB)], axis=1)
    rz = jnp.dot(wr_ref[...], xcat, preferred_element_type=jnp.float32)
    rz = rz + br_ref[...]
    for j in range(NB):
        r_ref[j] = rz[:, j * vtp:(j + 1) * vtp]
    rzm = rz * gap_ref[...]
    str_ref[0, :, 0:1] = jnp.sum(rzm, axis=1, keepdims=True)
    str_ref[0, :, 1:2] = jnp.sum(rzm * rzm, axis=1, keepdims=True)


def _finish_sgc(v_ref, res_ref, st_ref, g_ref, be_ref, m_ref, b_ref, gm_ref,
                x_ref, u_ref, sto_ref, xs_ref):
    c = x_ref.shape[1]
    tp = x_ref.shape[2] // V
    t = tp - 2 * PAD
    cnt = jnp.float32(st_ref.shape[0] * NB * t * V)
    scale, shift = _finalize(st_ref, g_ref, be_ref, cnt)
    for j in range(NB):
        pre = v_ref[j] * scale + shift
        if res_ref is not None:
            pre = pre + res_ref[j]
        xj = jnp.maximum(pre, 0.0)
        x_ref[j] = xj
        for vv_ in range(V):
            xs_ref[vv_ * c:(vv_ + 1) * c, j * t:(j + 1) * t] = (
                xj[:, vv_ * tp + PAD:vv_ * tp + PAD + t])
    u = jnp.dot(m_ref[...], xs_ref[...],
                preferred_element_type=jnp.float32) + b_ref[...]
    for j in range(NB):
        u_ref[j] = u[:, j * t:(j + 1) * t]
    _row_stats(u, gm_ref, sto_ref)


def _finish_sgc_body(v_ref, st_ref, g_ref, be_ref, m_ref, b_ref, gm_ref,
                     x_ref, u_ref, sto_ref, xs_ref):
    _finish_sgc(v_ref, None, st_ref, g_ref, be_ref, m_ref, b_ref, gm_ref,
                x_ref, u_ref, sto_ref, xs_ref)


def _finish_res_sgc_body(v_ref, st_ref, g_ref, be_ref, r_ref, m_ref, b_ref,
                         gm_ref, x_ref, u_ref, sto_ref, xs_ref):
    _finish_sgc(v_ref, r_ref, st_ref, g_ref, be_ref, m_ref, b_ref,
                gm_ref, x_ref, u_ref, sto_ref, xs_ref)


def _final_body(v_ref, stv_ref, gv_ref, bev_ref, r_ref, str_ref, gr_ref,
                ber_ref, o_ref):
    tp = v_ref.shape[2] // V
    t = tp - 2 * PAD
    cnt = jnp.float32(stv_ref.shape[0] * NB * t * V)
    sv, bv = _finalize(stv_ref, gv_ref, bev_ref, cnt)
    sr, br = _finalize(str_ref, gr_ref, ber_ref, cnt)
    for j in range(NB):
        o_ref[j] = jnp.maximum(v_ref[j] * sv + bv + r_ref[j] * sr + br, 0.0)


def _full(shape):
    return pl.BlockSpec(shape, lambda n: (0,) * len(shape))


def _pern(shape):
    return pl.BlockSpec((NB,) + shape, lambda n: (n, 0, 0))


def _pst(shape):
    return pl.BlockSpec((1,) + shape, lambda n: (n, 0, 0))


def _build_m(Wg, bg, A, edge, c_in, c_out):
    b = A * edge
    wr = Wg.reshape(K, c_out, c_in)
    m = jnp.einsum('kci,kvw->wcvi', wr, b).reshape(c_out * V, c_in * V)
    bias = jnp.einsum('kc,kw->wc', bg.reshape(K, c_out),
                      jnp.sum(b, axis=1)).reshape(c_out * V, 1)
    return m, bias


def _wstack(Wt):
    o, i, taps, _ = Wt.shape
    return Wt[:, :, :, 0].transpose(0, 2, 1).reshape(o, taps * i)


def _group(c):
    return jnp.tile(jnp.eye(c, dtype=jnp.float32), (1, V))


def kernel(x, A, g0, b0, Wg0, bg0, edge0, sg0, sb0, Wt0, bt0, tg0, tb0,
           Wg1, bg1, edge1, sg1, sb1, Wt1, bt1, tg1, tb1,
           Wg2, bg2, edge2, sg2, sb2, Wt2, bt2, tg2, tb2, Wr2, br2, rg2, rb2):
    n, c0, t, v = x.shape
    assert v == V and n % NB == 0
    steps = n // NB
    c1 = sg0.shape[0]
    c2 = sg1.shape[0]
    c3 = sg2.shape[0]
    tp = t + 2 * PAD
    vtp = v * tp
    f32 = jnp.float32
    col = lambda a: a.reshape(-1, 1)

    m0, bias0 = _build_m(Wg0, bg0, A, edge0, c0, c1)
    m1, bias1 = _build_m(Wg1, bg1, A, edge1, c1, c2)
    m2, bias2 = _build_m(Wg2, bg2, A, edge2, c2, c3)
    wt0 = _wstack(Wt0)
    wt1 = _wstack(Wt1)
    wt2 = _wstack(Wt2)
    g1 = _group(c1)
    g2 = _group(c2)
    g3 = _group(c3)

    tcol = jnp.arange(NB * vtp, dtype=jnp.int32) % tp
    gap = ((tcol >= PAD) & (tcol < PAD + t)).astype(f32)[None, :]

    x2d = x.reshape(n * c0, t * v)
    g_in = jnp.tile(jnp.eye(c0, dtype=f32), (1, n))
    st_in = pl.pallas_call(
        _stats_in_body,
        out_shape=jax.ShapeDtypeStruct((c0, 2), f32),
        in_specs=[pl.BlockSpec((n * c0, t * v), lambda: (0, 0)),
                  pl.BlockSpec((c0, n * c0), lambda: (0, 0))],
        out_specs=pl.BlockSpec((c0, 2), lambda: (0, 0)),
    )(x2d, g_in)

    xt = x.transpose(0, 3, 1, 2).reshape(n, v * c0, t)

    u0, st_u0 = pl.pallas_call(
        functools.partial(_sgc0_body, nbatch=n),
        grid=(steps,),
        out_shape=(jax.ShapeDtypeStruct((n, v * c1, t), f32),
                   jax.ShapeDtypeStruct((steps, c1, 2), f32)),
        in_specs=[_pern((v * c0, t)), _full((c0, 2)), _full((c0, 1)),
                  _full((c0, 1)), _full((v * c1, v * c0)), _full((v * c1, 1)),
                  _full((c1, v * c1))],
        out_specs=(_pern((v * c1, t)), _pst((c1, 2))),
    )(xt, st_in, col(g0), col(b0), m0, bias0, g1)

    v0, st_v0 = pl.pallas_call(
        _tconv_body,
        grid=(steps,),
        out_shape=(jax.ShapeDtypeStruct((n, c1, vtp), f32),
                   jax.ShapeDtypeStruct((steps, c1, 2), f32)),
        in_specs=[_pern((v * c1, t)), _full((steps, c1, 2)), _full((c1, 1)),
                  _full((c1, 1)), _full((c1, NTAP * c1)), _full((c1, 1)),
                  _full((1, NB * vtp))],
        out_specs=(_pern((c1, vtp)), _pst((c1, 2))),
        scratch_shapes=[pltpu.VMEM((c1, NB * vtp + 2 * PAD), f32)],
    )(u0, st_u0, col(sg0), col(sb0), wt0, col(bt0), gap)

    x1, u1, st_u1 = pl.pallas_call(
        _finish_sgc_body,
        grid=(steps,),
        out_shape=(jax.ShapeDtypeStruct((n, c1, vtp), f32),
                   jax.ShapeDtypeStruct((n, v * c2, t), f32),
                   jax.ShapeDtypeStruct((steps, c2, 2), f32)),
        in_specs=[_pern((c1, vtp)), _full((steps, c1, 2)), _full((c1, 1)),
                  _full((c1, 1)), _full((v * c2, v * c1)), _full((v * c2, 1)),
                  _full((c2, v * c2))],
        out_specs=(_pern((c1, vtp)), _pern((v * c2, t)), _pst((c2, 2))),
        scratch_shapes=[pltpu.VMEM((v * c1, NB * t), f32)],
    )(v0, st_v0, col(tg0), col(tb0), m1, bias1, g2)

    v1, st_v1 = pl.pallas_call(
        _tconv_body,
        grid=(steps,),
        out_shape=(jax.ShapeDtypeStruct((n, c2, vtp), f32),
                   jax.ShapeDtypeStruct((steps, c2, 2), f32)),
        in_specs=[_pern((v * c2, t)), _full((steps, c2, 2)), _full((c2, 1)),
                  _full((c2, 1)), _full((c2, NTAP * c2)), _full((c2, 1)),
                  _full((1, NB * vtp))],
        out_specs=(_pern((c2, vtp)), _pst((c2, 2))),
        scratch_shapes=[pltpu.VMEM((c2, NB * vtp + 2 * PAD), f32)],
    )(u1, st_u1, col(sg1), col(sb1), wt1, col(bt1), gap)

    x2, u2, st_u2 = pl.pallas_call(
        _finish_res_sgc_body,
        grid=(steps,),
        out_shape=(jax.ShapeDtypeStruct((n, c2, vtp), f32),
                   jax.ShapeDtypeStruct((n, v * c3, t), f32),
                   jax.ShapeDtypeStruct((steps, c3, 2), f32)),
        in_specs=[_pern((c2, vtp)), _full((steps, c2, 2)), _full((c2, 1)),
                  _full((c2, 1)), _pern((c2, vtp)),
                  _full((v * c3, v * c2)), _full((v * c3, 1)),
                  _full((c3, v * c3))],
        out_specs=(_pern((c2, vtp)), _pern((v * c3, t)), _pst((c3, 2))),
        scratch_shapes=[pltpu.VMEM((v * c2, NB * t), f32)],
    )(v1, st_v1, col(tg1), col(tb1), x1, m2, bias2, g3)

    v2, st_v2, r2, st_r2 = pl.pallas_call(
        _tconv_proj_body,
        grid=(steps,),
        out_shape=(jax.ShapeDtypeStruct((n, c3, vtp), f32),
                   jax.ShapeDtypeStruct((steps, c3, 2), f32),
                   jax.ShapeDtypeStruct((n, c3, vtp), f32),
                   jax.ShapeDtypeStruct((steps, c3, 2), f32)),
        in_specs=[_pern((v * c3, t)), _full((steps, c3, 2)), _full((c3, 1)),
                  _full((c3, 1)), _full((c3, NTAP * c3)), _full((c3, 1)),
                  _full((1, NB * vtp)), _pern((c2, vtp)), _full((c3, c2)),
                  _full((c3, 1))],
        out_specs=(_pern((c3, vtp)), _pst((c3, 2)),
                   _pern((c3, vtp)), _pst((c3, 2))),
        scratch_shapes=[pltpu.VMEM((c3, NB * vtp + 2 * PAD), f32)],
    )(u2, st_u2, col(sg2), col(sb2), wt2, col(bt2), gap, x2, Wr2, col(br2))

    out = pl.pallas_call(
        _final_body,
        grid=(steps,),
        out_shape=jax.ShapeDtypeStruct((n, c3, vtp), f32),
        in_specs=[_pern((c3, vtp)), _full((steps, c3, 2)), _full((c3, 1)),
                  _full((c3, 1)), _pern((c3, vtp)), _full((steps, c3, 2)),
                  _full((c3, 1)), _full((c3, 1))],
        out_specs=_pern((c3, vtp)),
    )(v2, st_v2, col(tg2), col(tb2), r2, st_r2, col(rg2), col(rb2))

    return (out.reshape(n, c3, v, tp)[:, :, :, PAD:PAD + t]
            .transpose(0, 1, 3, 2))
